# Initial kernel scaffold; baseline (speedup 1.0000x reference)
import jax
import jax.numpy as jnp
from jax.experimental import pallas as pl


def kernel(bias_high, bias_width, h_index, w_index):
    raise NotImplementedError("write your pallas kernel here")


# speedup vs baseline: 1.9899x; 1.9899x over previous
import jax
import jax.numpy as jnp
from jax import lax
from jax.experimental import pallas as pl
from jax.experimental.pallas import tpu as pltpu
from jax.experimental.pallas import tpu_sc as plsc

NUM_HEADS = 16
NREL = 66
TBL = NREL * NREL
NPIX = 1025 * 1025
BLK = 8 * 1025
NVEC = 513
TAIL_BASE = 128 * BLK
TAIL_N = NPIX - TAIL_BASE
TAIL_NVEC = 65
NC, NS, L = 2, 16, 16
COMB_WORDS = NUM_HEADS * TBL + 16


def _sc_body(bh_hbm, bw_hbm, hidx_hbm, widx_hbm, out_hbm,
             bh_v, bw_v, comb, hbuf, wbuf, obuf0, obuf1,
             sem_in, sem_o0, sem_o1):
    wid = lax.axis_index("s") * NC + lax.axis_index("c")

    pltpu.sync_copy(bh_hbm, bh_v)
    pltpu.sync_copy(bw_hbm, bw_v)
    for h in range(NUM_HEADS):
        row = [bw_v[pl.ds(h * 80 + v * 16, 16)] for v in range(5)]

        def build_a(a, _, h=h, row=row):
            s = plsc.load_gather(bh_v, [jnp.full((16,), h * 80 + a, jnp.int32)])
            base = h * TBL + a * NREL
            for v in range(5):
                comb[pl.ds(base + v * 16, 16)] = s + row[v]
            return 0

        lax.fori_loop(0, NREL, build_a, 0)

    obufs = (obuf0, obuf1)
    sems = (sem_o0, sem_o1)
    pending = [None, None]

    def do_block(base, n_dma, n_vec):
        cp1 = pltpu.async_copy(hidx_hbm.at[pl.ds(base, n_dma)],
                               hbuf.at[pl.ds(0, n_dma)], sem_in)
        cp2 = pltpu.async_copy(widx_hbm.at[pl.ds(base, n_dma)],
                               wbuf.at[pl.ds(0, n_dma)], sem_in)
        cp1.wait()
        cp2.wait()

        def cbody(v, _):
            off = v * 16
            c = hbuf[pl.ds(off, 16)] * NREL + wbuf[pl.ds(off, 16)]
            hbuf[pl.ds(off, 16)] = jnp.minimum(jnp.maximum(c, 0), TBL - 1)
            return 0

        lax.fori_loop(0, n_vec, cbody, 0)

        for h in range(NUM_HEADS):
            p = h % 2
            ob = obufs[p]
            if pending[p] is not None:
                pending[p].wait()
                pending[p] = None

            def gbody(v, _, h=h, ob=ob):
                off = v * 16
                cv = hbuf[pl.ds(off, 16)]
                ob[pl.ds(off, 16)] = plsc.load_gather(comb, [cv + h * TBL])
                return 0

            lax.fori_loop(0, n_vec, gbody, 0)
            pending[p] = pltpu.async_copy(
                ob.at[pl.ds(0, n_dma)],
                out_hbm.at[h, pl.ds(base, n_dma)], sems[p])

        for p in range(2):
            if pending[p] is not None:
                pending[p].wait()
                pending[p] = None

    for i in range(4):
        do_block((wid * 4 + i) * BLK, BLK, NVEC)

    @pl.when(wid == NC * NS - 1)
    def _tail():
        do_block(TAIL_BASE, TAIL_N, TAIL_NVEC)


def kernel(bias_high, bias_width, h_index, w_index):
    bh_t = jnp.zeros((NUM_HEADS, 80), jnp.float32).at[:, :NREL].set(
        bias_high.T).reshape(NUM_HEADS * 80)
    bw_t = jnp.zeros((NUM_HEADS, 80), jnp.float32).at[:, :NREL].set(
        bias_width.T).reshape(NUM_HEADS * 80)
    h_flat = h_index.reshape(NPIX).astype(jnp.int32)
    w_flat = w_index.reshape(NPIX).astype(jnp.int32)

    run = pl.kernel(
        _sc_body,
        out_type=jax.ShapeDtypeStruct((NUM_HEADS, NPIX), jnp.float32),
        mesh=plsc.VectorSubcoreMesh(core_axis_name="c", subcore_axis_name="s",
                                    num_cores=NC, num_subcores=NS),
        compiler_params=pltpu.CompilerParams(use_tc_tiling_on_sc=False,
                                             needs_layout_passes=False),
        scratch_types=[
            pltpu.VMEM((NUM_HEADS * 80,), jnp.float32),
            pltpu.VMEM((NUM_HEADS * 80,), jnp.float32),
            pltpu.VMEM((COMB_WORDS,), jnp.float32),
            pltpu.VMEM((NVEC * 16,), jnp.int32),
            pltpu.VMEM((NVEC * 16,), jnp.int32),
            pltpu.VMEM((NVEC * 16,), jnp.float32),
            pltpu.VMEM((NVEC * 16,), jnp.float32),
            pltpu.SemaphoreType.DMA,
            pltpu.SemaphoreType.DMA,
            pltpu.SemaphoreType.DMA,
        ],
    )
    out = run(bh_t, bw_t, h_flat, w_flat)
    return out.reshape(NUM_HEADS, 1025, 1025)
